# baseline (speedup 1.0000x reference)
import functools

import jax
import jax.numpy as jnp
from jax import lax
from jax.experimental import pallas as pl
from jax.experimental.pallas import tpu as pltpu
from jax.experimental.pallas import tpu_sc as plsc

N = 10000
DIM = 128
NUM_HEADS = 8
HEAD_DIM = DIM // NUM_HEADS
SCALE = HEAD_DIM ** (-0.5)

N_PAD = 10240
ROWS_PER_SUB = N_PAD // 16
DEN_ROWS = N_PAD // 16
CH = 32
NW = 32


def _qkv_body(h_ref, w_ref, q_ref, kv_ref):
    res = lax.dot_general(h_ref[...], w_ref[...], (((1,), (1,)), ((), ())),
                          preferred_element_type=jnp.float32,
                          precision=lax.Precision.HIGHEST)
    q_ref[...] = res[:, :DIM] * SCALE
    kv_ref[...] = res[:, DIM:]


def _qkv_project(h_pad, w_qkv):
    nblk = N_PAD // 512
    return pl.pallas_call(
        _qkv_body,
        grid=(nblk,),
        in_specs=[
            pl.BlockSpec((512, DIM), lambda i: (i, 0)),
            pl.BlockSpec((3 * DIM, DIM), lambda i: (0, 0)),
        ],
        out_specs=[
            pl.BlockSpec((512, DIM), lambda i: (i, 0)),
            pl.BlockSpec((512, 2 * DIM), lambda i: (i, 0)),
        ],
        out_shape=[
            jax.ShapeDtypeStruct((N_PAD, DIM), jnp.float32),
            jax.ShapeDtypeStruct((N_PAD, 2 * DIM), jnp.float32),
        ],
    )(h_pad, w_qkv)


def _edge_body(q_hbm, kv_hbm, epk_hbm, acc_out, den_out,
               idxb0, idxb1, sidx0, sidx1,
               sden0, sden1, pgo0, pgo1, qrows0, qrows1, kvrows0, kvrows1,
               msgw0, msgw1, denw0, denw1, acc_sh, den_sh,
               sem_gq0, sem_gq1, sem_gk0, sem_gk1,
               sem_sm0, sem_sm1, sem_sd0, sem_sd1, n_chunks):
    idxb = (idxb0, idxb1)
    pgo = (pgo0, pgo1)
    sidx = (sidx0, sidx1)
    sden = (sden0, sden1)
    qrows = (qrows0, qrows1)
    kvrows = (kvrows0, kvrows1)
    msgw = (msgw0, msgw1)
    denw = (denw0, denw1)
    sem_gq = (sem_gq0, sem_gq1)
    sem_gk = (sem_gk0, sem_gk1)
    sem_sm = (sem_sm0, sem_sm1)
    sem_sd = (sem_sd0, sem_sd1)

    c = lax.axis_index("c")
    s = lax.axis_index("s")
    wid = s * 2 + c
    zero16 = jnp.zeros((16,), jnp.float32)
    lane = lax.iota(jnp.int32, 16)
    lo_int = jnp.bitwise_and(lax.shift_right_logical(15 - lane, 3), 1)

    def _zrow(r, carry):
        for g in range(DIM // 16):
            msgw0[r, pl.ds(g * 16, 16)] = zero16
            denw0[r, pl.ds(g * 16, 16)] = zero16
            denw1[r, pl.ds(g * 16, 16)] = zero16
        pgo0[r] = 0
        pgo1[r] = 0
        return carry
    lax.fori_loop(0, CH, _zrow, 0)

    def _zacc(z, carry):
        pltpu.sync_copy(msgw0, acc_sh.at[pl.ds(s * ROWS_PER_SUB + z * CH, CH)])
        return carry
    lax.fori_loop(0, ROWS_PER_SUB // CH, _zacc, 0)
    dps = DEN_ROWS // 16
    pltpu.sync_copy(msgw0, den_sh.at[pl.ds(s * dps, CH)])
    pltpu.sync_copy(msgw0.at[pl.ds(0, dps - CH)],
                    den_sh.at[pl.ds(s * dps + CH, dps - CH)])
    plsc.subcore_barrier()

    edges_per_worker = n_chunks * CH

    def _stage(chunk, b):
        gchunk = wid * n_chunks + chunk
        pltpu.async_copy(epk_hbm.at[gchunk], idxb[b], sem_gq[b]).wait()
        pltpu.async_copy(q_hbm.at[idxb[b].at[0]], qrows[b], sem_gq[b])
        pltpu.async_copy(kv_hbm.at[idxb[b].at[1]], kvrows[b], sem_gk[b])

    _stage(0, 0)
    _stage(1, 1)

    def _iter(jj, carry):
        for b in range(2):
            pltpu.make_async_copy(q_hbm.at[idxb[b].at[0]], qrows[b],
                                  sem_gq[b]).wait()
            pltpu.make_async_copy(kv_hbm.at[idxb[b].at[1]], kvrows[b],
                                  sem_gk[b]).wait()

            @pl.when(jj > 0)
            def _wait_scatters():
                pltpu.make_async_copy(msgw[b], acc_sh.at[sidx[b]],
                                      sem_sm[b]).wait()
                pltpu.make_async_copy(denw[b], den_sh.at[sden[b]],
                                      sem_sd[b]).wait()

            for g in range(CH // 16):
                v = idxb[b][0, pl.ds(g * 16, 16)]
                sidx[b][pl.ds(g * 16, 16)] = v
                sden[b][pl.ds(g * 16, 16)] = lax.shift_right_logical(v, 4)

            def _group(gr, cy, b=b):
                dv = sidx[b][pl.ds(gr * 16, 16)]
                pv = jnp.bitwise_and(dv, 15)
                for li in range(16):
                    e = gr * 16 + li
                    sa = zero16
                    sb = zero16
                    for g in range(NUM_HEADS // 2):
                        sa = sa + (qrows[b][e, pl.ds(g * 16, 16)]
                                   * kvrows[b][e, pl.ds(g * 16, 16)])
                        g2 = g + NUM_HEADS // 2
                        sb = sb + (qrows[b][e, pl.ds(g2 * 16, 16)]
                                   * kvrows[b][e, pl.ds(g2 * 16, 16)])
                    s16 = sa + sb
                    ex = jnp.exp(s16 + lax.rev(s16, (0,)))
                    for g in range(NUM_HEADS):
                        vv = kvrows[b][e, pl.ds(DIM + g * 16, 16)]
                        msgw[b][e, pl.ds(g * 16, 16)] = vv * ex
                    p = pv[li]
                    oddv = jnp.full((16,), jnp.bitwise_and(p, 1),
                                    jnp.int32)
                    keep = jnp.bitwise_xor(lo_int, oddv)
                    val = ex * keep.astype(jnp.float32)
                    denw[b][e, pl.ds(pgo[b][e], 16)] = zero16
                    goff = lax.shift_left(lax.shift_right_logical(p, 1), 4)
                    denw[b][e, pl.ds(goff, 16)] = val
                    pgo[b][e] = goff
                return cy
            lax.fori_loop(0, CH // 16, _group, 0)

            pltpu.async_copy(msgw[b], acc_sh.at[sidx[b]], sem_sm[b],
                             add=True)
            pltpu.async_copy(denw[b], den_sh.at[sden[b]], sem_sd[b],
                             add=True)
            _stage(2 * jj + b + 2, b)
        return carry

    lax.fori_loop(0, n_chunks // 2, _iter, 0)

    for b in range(2):
        pltpu.make_async_copy(q_hbm.at[idxb[b].at[0]], qrows[b],
                              sem_gq[b]).wait()
        pltpu.make_async_copy(kv_hbm.at[idxb[b].at[1]], kvrows[b],
                              sem_gk[b]).wait()
        pltpu.make_async_copy(msgw[b], acc_sh.at[sidx[b]], sem_sm[b]).wait()
        pltpu.make_async_copy(denw[b], den_sh.at[sden[b]], sem_sd[b]).wait()
    plsc.subcore_barrier()

    pltpu.sync_copy(acc_sh.at[pl.ds(s * ROWS_PER_SUB, ROWS_PER_SUB)],
                    acc_out.at[c, pl.ds(s * ROWS_PER_SUB, ROWS_PER_SUB)])
    pltpu.sync_copy(den_sh.at[pl.ds(s * (DEN_ROWS // 16), DEN_ROWS // 16)],
                    den_out.at[c, pl.ds(s * (DEN_ROWS // 16), DEN_ROWS // 16)])


def _edge_aggregate(q, kv, epk, n_chunks):
    mesh = plsc.VectorSubcoreMesh(core_axis_name="c", subcore_axis_name="s")
    kfn = functools.partial(
        pl.kernel,
        mesh=mesh,
        out_type=[
            jax.ShapeDtypeStruct((2, N_PAD, DIM), jnp.float32),
            jax.ShapeDtypeStruct((2, DEN_ROWS, DIM), jnp.float32),
        ],
        scratch_types=(
            [pltpu.VMEM((2, CH), jnp.int32)] * 2
            + [pltpu.VMEM((CH,), jnp.int32)] * 4
            + [pltpu.SMEM((CH,), jnp.int32)] * 2
            + [pltpu.VMEM((CH, DIM), jnp.float32)] * 2
            + [pltpu.VMEM((CH, 2 * DIM), jnp.float32)] * 2
            + [pltpu.VMEM((CH, DIM), jnp.float32)] * 4
            + [
                pltpu.VMEM_SHARED((N_PAD, DIM), jnp.float32),
                pltpu.VMEM_SHARED((DEN_ROWS, DIM), jnp.float32),
            ]
            + [pltpu.SemaphoreType.DMA] * 8
        ),
    )(functools.partial(_edge_body, n_chunks=n_chunks))
    return kfn(q, kv, epk)


def _out_body(a0_ref, a1_ref, d0_ref, d1_ref, h_ref, w2_ref, wout_ref,
              b_ref, bmat_ref, rev_ref, o_ref):
    msg = a0_ref[0] + a1_ref[0]
    den = d0_ref[0] + d1_ref[0]
    par = lax.rem(lax.broadcasted_iota(jnp.int32, den.shape, 0), 2) == 1
    den_r = lax.dot_general(den, rev_ref[...], (((1,), (0,)), ((), ())),
                            preferred_element_type=jnp.float32,
                            precision=lax.Precision.HIGHEST)
    den = jnp.where(par, den_r, den)
    den = jnp.where(den == 0.0, 1.0, den)
    rr = lax.dot_general(1.0 / den, bmat_ref[...], (((1,), (0,)), ((), ())),
                         preferred_element_type=jnp.float32,
                         precision=lax.Precision.HIGHEST)
    hn = msg * rr
    out = lax.dot_general(hn, w2_ref[...], (((1,), (1,)), ((), ())),
                          preferred_element_type=jnp.float32,
                          precision=lax.Precision.HIGHEST)
    out += lax.dot_general(h_ref[...], wout_ref[...], (((1,), (1,)), ((), ())),
                           preferred_element_type=jnp.float32,
                           precision=lax.Precision.HIGHEST)
    o_ref[...] = out + b_ref[...]


def _merge_project(acc, den, h, w2, w_out, b_out, bmat, revm):
    nblk = N // 400
    return pl.pallas_call(
        _out_body,
        grid=(nblk,),
        in_specs=[
            pl.BlockSpec((1, 400, DIM), lambda i: (0, i, 0)),
            pl.BlockSpec((1, 400, DIM), lambda i: (1, i, 0)),
            pl.BlockSpec((1, 400, NUM_HEADS), lambda i: (0, i, 0)),
            pl.BlockSpec((1, 400, NUM_HEADS), lambda i: (1, i, 0)),
            pl.BlockSpec((400, DIM), lambda i: (i, 0)),
            pl.BlockSpec((DIM, DIM), lambda i: (0, 0)),
            pl.BlockSpec((DIM, DIM), lambda i: (0, 0)),
            pl.BlockSpec((1, DIM), lambda i: (0, 0)),
            pl.BlockSpec((NUM_HEADS, DIM), lambda i: (0, 0)),
            pl.BlockSpec((NUM_HEADS, NUM_HEADS), lambda i: (0, 0)),
        ],
        out_specs=pl.BlockSpec((400, DIM), lambda i: (i, 0)),
        out_shape=jax.ShapeDtypeStruct((N, DIM), jnp.float32),
    )(acc, acc, den, den, h, w2, w_out, b_out, bmat, revm)


def kernel(h, edge_index, W_qkv, W_out, b_out):
    h = h.astype(jnp.float32)
    e_total = edge_index.shape[1]
    n_chunks = -(-e_total // (NW * CH))
    n_chunks += n_chunks % 2
    e_alloc = NW * CH * n_chunks + 2 * CH

    h_pad = jnp.pad(h, ((0, N_PAD - N), (0, 0)))
    src_p = jnp.pad(edge_index[0].astype(jnp.int32), (0, e_alloc - e_total),
                    constant_values=N)
    dst_p = jnp.pad(edge_index[1].astype(jnp.int32), (0, e_alloc - e_total),
                    constant_values=N)
    epk = jnp.stack([dst_p.reshape(-1, CH), src_p.reshape(-1, CH)], axis=1)

    col = jnp.arange(DIM)
    jj, l = col // 16, col % 16
    head = jnp.where(l < 8, l, 15 - l)
    dim = 2 * jj + (l >= 8)
    perm = head * HEAD_DIM + dim
    wq = W_qkv.astype(jnp.float32)
    w_perm = jnp.concatenate([wq[:DIM][perm], wq[DIM:2 * DIM][perm],
                              wq[2 * DIM:][perm]], axis=0)

    q, kv = _qkv_project(h_pad, w_perm)
    acc, den = _edge_aggregate(q, kv, epk, n_chunks)
    den = den.reshape(2, N_PAD, NUM_HEADS)

    w2 = W_out[:, dim * NUM_HEADS + head].astype(jnp.float32)
    bmat = (head[None, :] == jnp.arange(NUM_HEADS)[:, None]).astype(
        jnp.float32)
    revm = jnp.eye(NUM_HEADS, dtype=jnp.float32)[::-1]
    return _merge_project(acc, den, h, w2, W_out.astype(jnp.float32),
                          b_out.reshape(1, DIM).astype(jnp.float32), bmat,
                          revm)


# speedup vs baseline: 1.0463x; 1.0463x over previous
import functools

import jax
import jax.numpy as jnp
from jax import lax
from jax.experimental import pallas as pl
from jax.experimental.pallas import tpu as pltpu
from jax.experimental.pallas import tpu_sc as plsc

N = 10000
DIM = 128
NUM_HEADS = 8
HEAD_DIM = DIM // NUM_HEADS
SCALE = HEAD_DIM ** (-0.5)

N_PAD = 10240
ROWS_PER_SUB = N_PAD // 16
DEN_ROWS = N_PAD // 16
CH = 32
NW = 32


def _qkv_body(h_ref, w_ref, q_ref, kv_ref):
    res = lax.dot_general(h_ref[...], w_ref[...], (((1,), (1,)), ((), ())),
                          preferred_element_type=jnp.float32,
                          precision=lax.Precision.HIGHEST)
    q_ref[...] = res[:, :DIM] * SCALE
    kv_ref[...] = res[:, DIM:]


def _qkv_project(h_pad, w_qkv):
    nblk = N_PAD // 512
    return pl.pallas_call(
        _qkv_body,
        grid=(nblk,),
        in_specs=[
            pl.BlockSpec((512, DIM), lambda i: (i, 0)),
            pl.BlockSpec((3 * DIM, DIM), lambda i: (0, 0)),
        ],
        out_specs=[
            pl.BlockSpec((512, DIM), lambda i: (i, 0)),
            pl.BlockSpec((512, 2 * DIM), lambda i: (i, 0)),
        ],
        out_shape=[
            jax.ShapeDtypeStruct((N_PAD, DIM), jnp.float32),
            jax.ShapeDtypeStruct((N_PAD, 2 * DIM), jnp.float32),
        ],
    )(h_pad, w_qkv)


def _edge_body(q_hbm, kv_hbm, src_hbm, dst_hbm, acc_out, den_out,
               idx_dst0, idx_dst1, idx_src0, idx_src1, sidx0, sidx1,
               sden0, sden1, qrows0, qrows1, kvrows0, kvrows1,
               msgw0, msgw1, denw0, denw1, acc_sh, den_sh,
               sem_gq0, sem_gq1, sem_gk0, sem_gk1,
               sem_sm0, sem_sm1, sem_sd0, sem_sd1, n_chunks):
    idx_dst = (idx_dst0, idx_dst1)
    idx_src = (idx_src0, idx_src1)
    sidx = (sidx0, sidx1)
    sden = (sden0, sden1)
    qrows = (qrows0, qrows1)
    kvrows = (kvrows0, kvrows1)
    msgw = (msgw0, msgw1)
    denw = (denw0, denw1)
    sem_gq = (sem_gq0, sem_gq1)
    sem_gk = (sem_gk0, sem_gk1)
    sem_sm = (sem_sm0, sem_sm1)
    sem_sd = (sem_sd0, sem_sd1)

    c = lax.axis_index("c")
    s = lax.axis_index("s")
    wid = s * 2 + c
    zero16 = jnp.zeros((16,), jnp.float32)
    lane = lax.iota(jnp.int32, 16)
    lo_int = jnp.bitwise_and(lax.shift_right_logical(15 - lane, 3), 1)

    def _zrow(r, carry):
        for g in range(DIM // 16):
            msgw0[r, pl.ds(g * 16, 16)] = zero16
        return carry
    lax.fori_loop(0, CH, _zrow, 0)

    def _zacc(z, carry):
        pltpu.sync_copy(msgw0, acc_sh.at[pl.ds(s * ROWS_PER_SUB + z * CH, CH)])
        return carry
    lax.fori_loop(0, ROWS_PER_SUB // CH, _zacc, 0)
    dps = DEN_ROWS // 16
    pltpu.sync_copy(msgw0, den_sh.at[pl.ds(s * dps, CH)])
    pltpu.sync_copy(msgw0.at[pl.ds(0, dps - CH)],
                    den_sh.at[pl.ds(s * dps + CH, dps - CH)])
    plsc.subcore_barrier()

    edges_per_worker = n_chunks * CH

    def _stage(chunk, b):
        off = wid * edges_per_worker + chunk * CH
        i1 = pltpu.async_copy(dst_hbm.at[pl.ds(off, CH)], idx_dst[b],
                              sem_gq[b])
        i2 = pltpu.async_copy(src_hbm.at[pl.ds(off, CH)], idx_src[b],
                              sem_gk[b])
        i1.wait()
        i2.wait()
        pltpu.async_copy(q_hbm.at[idx_dst[b]], qrows[b], sem_gq[b])
        pltpu.async_copy(kv_hbm.at[idx_src[b]], kvrows[b], sem_gk[b])

    _stage(0, 0)
    _stage(1, 1)

    def _iter(jj, carry):
        for b in range(2):
            pltpu.make_async_copy(q_hbm.at[idx_dst[b]], qrows[b],
                                  sem_gq[b]).wait()
            pltpu.make_async_copy(kv_hbm.at[idx_src[b]], kvrows[b],
                                  sem_gk[b]).wait()

            @pl.when(jj > 0)
            def _wait_scatters():
                pltpu.make_async_copy(msgw[b], acc_sh.at[sidx[b]],
                                      sem_sm[b]).wait()
                pltpu.make_async_copy(denw[b], den_sh.at[sden[b]],
                                      sem_sd[b]).wait()

            for g in range(CH // 16):
                v = idx_dst[b][pl.ds(g * 16, 16)]
                sidx[b][pl.ds(g * 16, 16)] = v
                sden[b][pl.ds(g * 16, 16)] = lax.shift_right_logical(v, 4)

            def _group(gr, cy, b=b):
                dv = sidx[b][pl.ds(gr * 16, 16)]
                pv = jnp.bitwise_and(dv, 15)
                for li in range(16):
                    e = gr * 16 + li
                    sa = zero16
                    sb = zero16
                    for g in range(NUM_HEADS // 2):
                        sa = sa + (qrows[b][e, pl.ds(g * 16, 16)]
                                   * kvrows[b][e, pl.ds(g * 16, 16)])
                        g2 = g + NUM_HEADS // 2
                        sb = sb + (qrows[b][e, pl.ds(g2 * 16, 16)]
                                   * kvrows[b][e, pl.ds(g2 * 16, 16)])
                    s16 = sa + sb
                    ex = jnp.exp(s16 + lax.rev(s16, (0,)))
                    for g in range(NUM_HEADS):
                        vv = kvrows[b][e, pl.ds(DIM + g * 16, 16)]
                        msgw[b][e, pl.ds(g * 16, 16)] = vv * ex
                    p = pv[li]
                    oddv = jnp.full((16,), jnp.bitwise_and(p, 1),
                                    jnp.int32)
                    keep = jnp.bitwise_xor(lo_int, oddv)
                    val = ex * keep.astype(jnp.float32)
                    for g in range(NUM_HEADS):
                        denw[b][e, pl.ds(g * 16, 16)] = zero16
                    goff = lax.shift_left(lax.shift_right_logical(p, 1), 4)
                    denw[b][e, pl.ds(goff, 16)] = val
                return cy
            lax.fori_loop(0, CH // 16, _group, 0)

            pltpu.async_copy(msgw[b], acc_sh.at[sidx[b]], sem_sm[b],
                             add=True)
            pltpu.async_copy(denw[b], den_sh.at[sden[b]], sem_sd[b],
                             add=True)
            _stage(2 * jj + b + 2, b)
        return carry

    lax.fori_loop(0, n_chunks // 2, _iter, 0)

    for b in range(2):
        pltpu.make_async_copy(q_hbm.at[idx_dst[b]], qrows[b],
                              sem_gq[b]).wait()
        pltpu.make_async_copy(kv_hbm.at[idx_src[b]], kvrows[b],
                              sem_gk[b]).wait()
        pltpu.make_async_copy(msgw[b], acc_sh.at[sidx[b]], sem_sm[b]).wait()
        pltpu.make_async_copy(denw[b], den_sh.at[sden[b]], sem_sd[b]).wait()
    plsc.subcore_barrier()

    pltpu.sync_copy(acc_sh.at[pl.ds(s * ROWS_PER_SUB, ROWS_PER_SUB)],
                    acc_out.at[c, pl.ds(s * ROWS_PER_SUB, ROWS_PER_SUB)])
    pltpu.sync_copy(den_sh.at[pl.ds(s * (DEN_ROWS // 16), DEN_ROWS // 16)],
                    den_out.at[c, pl.ds(s * (DEN_ROWS // 16), DEN_ROWS // 16)])


def _edge_aggregate(q, kv, src_p, dst_p, n_chunks):
    mesh = plsc.VectorSubcoreMesh(core_axis_name="c", subcore_axis_name="s")
    kfn = functools.partial(
        pl.kernel,
        mesh=mesh,
        out_type=[
            jax.ShapeDtypeStruct((2, N_PAD, DIM), jnp.float32),
            jax.ShapeDtypeStruct((2, DEN_ROWS, DIM), jnp.float32),
        ],
        scratch_types=(
            [pltpu.VMEM((CH,), jnp.int32)] * 8
            + [pltpu.VMEM((CH, DIM), jnp.float32)] * 2
            + [pltpu.VMEM((CH, 2 * DIM), jnp.float32)] * 2
            + [pltpu.VMEM((CH, DIM), jnp.float32)] * 4
            + [
                pltpu.VMEM_SHARED((N_PAD, DIM), jnp.float32),
                pltpu.VMEM_SHARED((DEN_ROWS, DIM), jnp.float32),
            ]
            + [pltpu.SemaphoreType.DMA] * 8
        ),
    )(functools.partial(_edge_body, n_chunks=n_chunks))
    return kfn(q, kv, src_p, dst_p)


def _out_body(a0_ref, a1_ref, d0_ref, d1_ref, h_ref, w2_ref, wout_ref,
              b_ref, bmat_ref, rev_ref, o_ref):
    msg = a0_ref[0] + a1_ref[0]
    den = d0_ref[0] + d1_ref[0]
    par = lax.rem(lax.broadcasted_iota(jnp.int32, den.shape, 0), 2) == 1
    den_r = lax.dot_general(den, rev_ref[...], (((1,), (0,)), ((), ())),
                            preferred_element_type=jnp.float32,
                            precision=lax.Precision.HIGHEST)
    den = jnp.where(par, den_r, den)
    den = jnp.where(den == 0.0, 1.0, den)
    rr = lax.dot_general(1.0 / den, bmat_ref[...], (((1,), (0,)), ((), ())),
                         preferred_element_type=jnp.float32,
                         precision=lax.Precision.HIGHEST)
    hn = msg * rr
    out = lax.dot_general(hn, w2_ref[...], (((1,), (1,)), ((), ())),
                          preferred_element_type=jnp.float32,
                          precision=lax.Precision.HIGHEST)
    out += lax.dot_general(h_ref[...], wout_ref[...], (((1,), (1,)), ((), ())),
                           preferred_element_type=jnp.float32,
                           precision=lax.Precision.HIGHEST)
    o_ref[...] = out + b_ref[...]


def _merge_project(acc, den, h, w2, w_out, b_out, bmat, revm):
    nblk = N // 400
    return pl.pallas_call(
        _out_body,
        grid=(nblk,),
        in_specs=[
            pl.BlockSpec((1, 400, DIM), lambda i: (0, i, 0)),
            pl.BlockSpec((1, 400, DIM), lambda i: (1, i, 0)),
            pl.BlockSpec((1, 400, NUM_HEADS), lambda i: (0, i, 0)),
            pl.BlockSpec((1, 400, NUM_HEADS), lambda i: (1, i, 0)),
            pl.BlockSpec((400, DIM), lambda i: (i, 0)),
            pl.BlockSpec((DIM, DIM), lambda i: (0, 0)),
            pl.BlockSpec((DIM, DIM), lambda i: (0, 0)),
            pl.BlockSpec((1, DIM), lambda i: (0, 0)),
            pl.BlockSpec((NUM_HEADS, DIM), lambda i: (0, 0)),
            pl.BlockSpec((NUM_HEADS, NUM_HEADS), lambda i: (0, 0)),
        ],
        out_specs=pl.BlockSpec((400, DIM), lambda i: (i, 0)),
        out_shape=jax.ShapeDtypeStruct((N, DIM), jnp.float32),
    )(acc, acc, den, den, h, w2, w_out, b_out, bmat, revm)


def kernel(h, edge_index, W_qkv, W_out, b_out):
    h = h.astype(jnp.float32)
    e_total = edge_index.shape[1]
    n_chunks = -(-e_total // (NW * CH))
    n_chunks += n_chunks % 2
    e_alloc = NW * CH * n_chunks + 2 * CH

    h_pad = jnp.pad(h, ((0, N_PAD - N), (0, 0)))
    src_p = jnp.pad(edge_index[0].astype(jnp.int32), (0, e_alloc - e_total),
                    constant_values=N)
    dst_p = jnp.pad(edge_index[1].astype(jnp.int32), (0, e_alloc - e_total),
                    constant_values=N)

    col = jnp.arange(DIM)
    jj, l = col // 16, col % 16
    head = jnp.where(l < 8, l, 15 - l)
    dim = 2 * jj + (l >= 8)
    perm = head * HEAD_DIM + dim
    wq = W_qkv.astype(jnp.float32)
    w_perm = jnp.concatenate([wq[:DIM][perm], wq[DIM:2 * DIM][perm],
                              wq[2 * DIM:][perm]], axis=0)

    q, kv = _qkv_project(h_pad, w_perm)
    acc, den = _edge_aggregate(q, kv, src_p, dst_p, n_chunks)
    den = den.reshape(2, N_PAD, NUM_HEADS)

    w2 = W_out[:, dim * NUM_HEADS + head].astype(jnp.float32)
    bmat = (head[None, :] == jnp.arange(NUM_HEADS)[:, None]).astype(
        jnp.float32)
    revm = jnp.eye(NUM_HEADS, dtype=jnp.float32)[::-1]
    return _merge_project(acc, den, h, w2, W_out.astype(jnp.float32),
                          b_out.reshape(1, DIM).astype(jnp.float32), bmat,
                          revm)


# speedup vs baseline: 1.1136x; 1.0643x over previous
import functools

import jax
import jax.numpy as jnp
from jax import lax
from jax.experimental import pallas as pl
from jax.experimental.pallas import tpu as pltpu
from jax.experimental.pallas import tpu_sc as plsc

N = 10000
DIM = 128
NUM_HEADS = 8
HEAD_DIM = DIM // NUM_HEADS
SCALE = HEAD_DIM ** (-0.5)

N_PAD = 10240
ROWS_PER_SUB = N_PAD // 16
DEN_ROWS = N_PAD // 16
CH = 32
NW = 32


def _qkv_body(h_ref, w_ref, q_ref, kv_ref):
    res = lax.dot_general(h_ref[...], w_ref[...], (((1,), (1,)), ((), ())),
                          preferred_element_type=jnp.float32,
                          precision=lax.Precision.HIGHEST)
    q_ref[...] = res[:, :DIM] * SCALE
    kv_ref[...] = res[:, DIM:]


def _qkv_project(h_pad, w_qkv):
    nblk = N_PAD // 512
    return pl.pallas_call(
        _qkv_body,
        grid=(nblk,),
        in_specs=[
            pl.BlockSpec((512, DIM), lambda i: (i, 0)),
            pl.BlockSpec((3 * DIM, DIM), lambda i: (0, 0)),
        ],
        out_specs=[
            pl.BlockSpec((512, DIM), lambda i: (i, 0)),
            pl.BlockSpec((512, 2 * DIM), lambda i: (i, 0)),
        ],
        out_shape=[
            jax.ShapeDtypeStruct((N_PAD, DIM), jnp.float32),
            jax.ShapeDtypeStruct((N_PAD, 2 * DIM), jnp.float32),
        ],
    )(h_pad, w_qkv)


def _edge_body(q_hbm, kv_hbm, src_hbm, dst_hbm, acc_out, den_out,
               idx_dst0, idx_dst1, idx_src0, idx_src1, sidx0, sidx1,
               sden0, sden1, qrows0, qrows1, kvrows0, kvrows1,
               msgw0, msgw1, denw0, denw1, acc_sh, den_sh,
               sem_gq0, sem_gq1, sem_gk0, sem_gk1,
               sem_sm0, sem_sm1, sem_sd0, sem_sd1, n_chunks):
    idx_dst = (idx_dst0, idx_dst1)
    idx_src = (idx_src0, idx_src1)
    sidx = (sidx0, sidx1)
    sden = (sden0, sden1)
    qrows = (qrows0, qrows1)
    kvrows = (kvrows0, kvrows1)
    msgw = (msgw0, msgw1)
    denw = (denw0, denw1)
    sem_gq = (sem_gq0, sem_gq1)
    sem_gk = (sem_gk0, sem_gk1)
    sem_sm = (sem_sm0, sem_sm1)
    sem_sd = (sem_sd0, sem_sd1)

    c = lax.axis_index("c")
    s = lax.axis_index("s")
    wid = s * 2 + c
    zero16 = jnp.zeros((16,), jnp.float32)
    lane = lax.iota(jnp.int32, 16)
    lo_int = jnp.bitwise_and(lax.shift_right_logical(15 - lane, 3), 1)

    def _zrow(r, carry):
        for g in range(DIM // 16):
            msgw0[r, pl.ds(g * 16, 16)] = zero16
        return carry
    lax.fori_loop(0, CH, _zrow, 0)

    def _zacc(z, carry):
        pltpu.sync_copy(msgw0, acc_sh.at[pl.ds(s * ROWS_PER_SUB + z * CH, CH)])
        return carry
    lax.fori_loop(0, ROWS_PER_SUB // CH, _zacc, 0)
    dps = DEN_ROWS // 16
    pltpu.sync_copy(msgw0, den_sh.at[pl.ds(s * dps, CH)])
    pltpu.sync_copy(msgw0.at[pl.ds(0, dps - CH)],
                    den_sh.at[pl.ds(s * dps + CH, dps - CH)])
    plsc.subcore_barrier()

    edges_per_worker = n_chunks * CH

    def _stage(chunk, b):
        off = wid * edges_per_worker + chunk * CH
        i1 = pltpu.async_copy(dst_hbm.at[pl.ds(off, CH)], idx_dst[b],
                              sem_gq[b])
        i2 = pltpu.async_copy(src_hbm.at[pl.ds(off, CH)], idx_src[b],
                              sem_gk[b])
        i1.wait()
        i2.wait()
        pltpu.async_copy(q_hbm.at[idx_dst[b]], qrows[b], sem_gq[b])
        pltpu.async_copy(kv_hbm.at[idx_src[b]], kvrows[b], sem_gk[b])

    _stage(0, 0)
    _stage(1, 1)

    def _iter(jj, carry):
        for b in range(2):
            pltpu.make_async_copy(q_hbm.at[idx_dst[b]], qrows[b],
                                  sem_gq[b]).wait()
            pltpu.make_async_copy(kv_hbm.at[idx_src[b]], kvrows[b],
                                  sem_gk[b]).wait()

            @pl.when(jj > 0)
            def _wait_scatters():
                pltpu.make_async_copy(msgw[b], acc_sh.at[sidx[b]],
                                      sem_sm[b]).wait()
                pltpu.make_async_copy(denw[b], den_sh.at[sden[b]],
                                      sem_sd[b]).wait()

            for g in range(CH // 16):
                v = idx_dst[b][pl.ds(g * 16, 16)]
                sidx[b][pl.ds(g * 16, 16)] = v
                sden[b][pl.ds(g * 16, 16)] = lax.shift_right_logical(v, 4)

            @plsc.parallel_loop(0, CH // 16, unroll=2)
            def _group(gr, b=b):
                dv = sidx[b][pl.ds(gr * 16, 16)]
                pv = jnp.bitwise_and(dv, 15)
                for li in range(16):
                    e = gr * 16 + li
                    s16 = zero16
                    for g in range(NUM_HEADS):
                        qv = qrows[b][e, pl.ds(g * 16, 16)]
                        kv = kvrows[b][e, pl.ds(g * 16, 16)]
                        s16 = s16 + qv * kv
                    ex = jnp.exp(s16 + lax.rev(s16, (0,)))
                    for g in range(NUM_HEADS):
                        vv = kvrows[b][e, pl.ds(DIM + g * 16, 16)]
                        msgw[b][e, pl.ds(g * 16, 16)] = vv * ex
                    p = pv[li]
                    oddv = jnp.full((16,), jnp.bitwise_and(p, 1),
                                    jnp.int32)
                    keep = jnp.bitwise_xor(lo_int, oddv)
                    val = ex * keep.astype(jnp.float32)
                    for g in range(NUM_HEADS):
                        denw[b][e, pl.ds(g * 16, 16)] = zero16
                    goff = lax.shift_left(lax.shift_right_logical(p, 1), 4)
                    denw[b][e, pl.ds(goff, 16)] = val

            pltpu.async_copy(msgw[b], acc_sh.at[sidx[b]], sem_sm[b],
                             add=True)
            pltpu.async_copy(denw[b], den_sh.at[sden[b]], sem_sd[b],
                             add=True)
            _stage(2 * jj + b + 2, b)
        return carry

    lax.fori_loop(0, n_chunks // 2, _iter, 0)

    for b in range(2):
        pltpu.make_async_copy(q_hbm.at[idx_dst[b]], qrows[b],
                              sem_gq[b]).wait()
        pltpu.make_async_copy(kv_hbm.at[idx_src[b]], kvrows[b],
                              sem_gk[b]).wait()
        pltpu.make_async_copy(msgw[b], acc_sh.at[sidx[b]], sem_sm[b]).wait()
        pltpu.make_async_copy(denw[b], den_sh.at[sden[b]], sem_sd[b]).wait()
    plsc.subcore_barrier()

    pltpu.sync_copy(acc_sh.at[pl.ds(s * ROWS_PER_SUB, ROWS_PER_SUB)],
                    acc_out.at[c, pl.ds(s * ROWS_PER_SUB, ROWS_PER_SUB)])
    pltpu.sync_copy(den_sh.at[pl.ds(s * (DEN_ROWS // 16), DEN_ROWS // 16)],
                    den_out.at[c, pl.ds(s * (DEN_ROWS // 16), DEN_ROWS // 16)])


def _edge_aggregate(q, kv, src_p, dst_p, n_chunks):
    mesh = plsc.VectorSubcoreMesh(core_axis_name="c", subcore_axis_name="s")
    kfn = functools.partial(
        pl.kernel,
        mesh=mesh,
        out_type=[
            jax.ShapeDtypeStruct((2, N_PAD, DIM), jnp.float32),
            jax.ShapeDtypeStruct((2, DEN_ROWS, DIM), jnp.float32),
        ],
        scratch_types=(
            [pltpu.VMEM((CH,), jnp.int32)] * 8
            + [pltpu.VMEM((CH, DIM), jnp.float32)] * 2
            + [pltpu.VMEM((CH, 2 * DIM), jnp.float32)] * 2
            + [pltpu.VMEM((CH, DIM), jnp.float32)] * 4
            + [
                pltpu.VMEM_SHARED((N_PAD, DIM), jnp.float32),
                pltpu.VMEM_SHARED((DEN_ROWS, DIM), jnp.float32),
            ]
            + [pltpu.SemaphoreType.DMA] * 8
        ),
    )(functools.partial(_edge_body, n_chunks=n_chunks))
    return kfn(q, kv, src_p, dst_p)


def _out_body(a0_ref, a1_ref, d0_ref, d1_ref, h_ref, w2_ref, wout_ref,
              b_ref, bmat_ref, rev_ref, o_ref):
    msg = a0_ref[0] + a1_ref[0]
    den = d0_ref[0] + d1_ref[0]
    par = lax.rem(lax.broadcasted_iota(jnp.int32, den.shape, 0), 2) == 1
    den_r = lax.dot_general(den, rev_ref[...], (((1,), (0,)), ((), ())),
                            preferred_element_type=jnp.float32,
                            precision=lax.Precision.HIGHEST)
    den = jnp.where(par, den_r, den)
    den = jnp.where(den == 0.0, 1.0, den)
    rr = lax.dot_general(1.0 / den, bmat_ref[...], (((1,), (0,)), ((), ())),
                         preferred_element_type=jnp.float32,
                         precision=lax.Precision.HIGHEST)
    hn = msg * rr
    out = lax.dot_general(hn, w2_ref[...], (((1,), (1,)), ((), ())),
                          preferred_element_type=jnp.float32,
                          precision=lax.Precision.HIGHEST)
    out += lax.dot_general(h_ref[...], wout_ref[...], (((1,), (1,)), ((), ())),
                           preferred_element_type=jnp.float32,
                           precision=lax.Precision.HIGHEST)
    o_ref[...] = out + b_ref[...]


def _merge_project(acc, den, h, w2, w_out, b_out, bmat, revm):
    nblk = N // 400
    return pl.pallas_call(
        _out_body,
        grid=(nblk,),
        in_specs=[
            pl.BlockSpec((1, 400, DIM), lambda i: (0, i, 0)),
            pl.BlockSpec((1, 400, DIM), lambda i: (1, i, 0)),
            pl.BlockSpec((1, 400, NUM_HEADS), lambda i: (0, i, 0)),
            pl.BlockSpec((1, 400, NUM_HEADS), lambda i: (1, i, 0)),
            pl.BlockSpec((400, DIM), lambda i: (i, 0)),
            pl.BlockSpec((DIM, DIM), lambda i: (0, 0)),
            pl.BlockSpec((DIM, DIM), lambda i: (0, 0)),
            pl.BlockSpec((1, DIM), lambda i: (0, 0)),
            pl.BlockSpec((NUM_HEADS, DIM), lambda i: (0, 0)),
            pl.BlockSpec((NUM_HEADS, NUM_HEADS), lambda i: (0, 0)),
        ],
        out_specs=pl.BlockSpec((400, DIM), lambda i: (i, 0)),
        out_shape=jax.ShapeDtypeStruct((N, DIM), jnp.float32),
    )(acc, acc, den, den, h, w2, w_out, b_out, bmat, revm)


def kernel(h, edge_index, W_qkv, W_out, b_out):
    h = h.astype(jnp.float32)
    e_total = edge_index.shape[1]
    n_chunks = -(-e_total // (NW * CH))
    n_chunks += n_chunks % 2
    e_alloc = NW * CH * n_chunks + 2 * CH

    h_pad = jnp.pad(h, ((0, N_PAD - N), (0, 0)))
    src_p = jnp.pad(edge_index[0].astype(jnp.int32), (0, e_alloc - e_total),
                    constant_values=N)
    dst_p = jnp.pad(edge_index[1].astype(jnp.int32), (0, e_alloc - e_total),
                    constant_values=N)

    col = jnp.arange(DIM)
    jj, l = col // 16, col % 16
    head = jnp.where(l < 8, l, 15 - l)
    dim = 2 * jj + (l >= 8)
    perm = head * HEAD_DIM + dim
    wq = W_qkv.astype(jnp.float32)
    w_perm = jnp.concatenate([wq[:DIM][perm], wq[DIM:2 * DIM][perm],
                              wq[2 * DIM:][perm]], axis=0)

    q, kv = _qkv_project(h_pad, w_perm)
    acc, den = _edge_aggregate(q, kv, src_p, dst_p, n_chunks)
    den = den.reshape(2, N_PAD, NUM_HEADS)

    w2 = W_out[:, dim * NUM_HEADS + head].astype(jnp.float32)
    bmat = (head[None, :] == jnp.arange(NUM_HEADS)[:, None]).astype(
        jnp.float32)
    revm = jnp.eye(NUM_HEADS, dtype=jnp.float32)[::-1]
    return _merge_project(acc, den, h, w2, W_out.astype(jnp.float32),
                          b_out.reshape(1, DIM).astype(jnp.float32), bmat,
                          revm)
